# Initial kernel scaffold; baseline (speedup 1.0000x reference)
import jax
import jax.numpy as jnp
from jax.experimental import pallas as pl


def kernel(x, Ae, be, Ad, bd):
    raise NotImplementedError("write your pallas kernel here")


# speedup vs baseline: 6.1846x; 6.1846x over previous
import functools

import jax
import jax.numpy as jnp
from jax.experimental import pallas as pl
from jax.experimental.pallas import tpu as pltpu

K = 64


def _to_key(h):
    u = jax.lax.bitcast_convert_type(h, jnp.uint32)
    return u ^ jnp.where(h < 0, jnp.uint32(0xFFFFFFFF), jnp.uint32(0x80000000))


def _body(x_ref, ae_ref, ad_ref, be_ref, bd_ref, out_ref, key_ref, thr_ref,
          acc_ref, *, nw, wb):
    p = pl.program_id(1)
    w = pl.program_id(2)

    @pl.when(p == 0)
    def _encode():
        xc = x_ref[...] - bd_ref[...]
        h = jax.lax.dot_general(
            xc, ae_ref[...],
            dimension_numbers=(((1,), (1,)), ((), ())),
            preferred_element_type=jnp.float32,
        ) + be_ref[...]
        key_ref[:, pl.ds(w * wb, wb)] = _to_key(h)

    @pl.when((p == 1) & (w == 0))
    def _threshold():
        keys = key_ref[...]
        t = jnp.zeros((keys.shape[0], 1), jnp.uint32)
        for b in range(31, -1, -1):
            cand = t | jnp.uint32(1 << b)
            cnt = jnp.sum((keys >= cand).astype(jnp.int32), axis=1,
                          keepdims=True)
            t = jnp.where(cnt >= K, cand, t)
        thr_ref[...] = t

    @pl.when(p == 1)
    def _decode():
        @pl.when(w == 0)
        def _init():
            acc_ref[...] = jnp.zeros_like(acc_ref)

        keys = key_ref[:, pl.ds(w * wb, wb)]
        t = thr_ref[...]
        sel = (keys >= t) & (keys > jnp.uint32(0x80000000))
        vals = jax.lax.bitcast_convert_type(keys ^ jnp.uint32(0x80000000),
                                            jnp.float32)
        codes = jnp.where(sel, vals, 0.0)
        acc_ref[...] += jax.lax.dot_general(
            codes, ad_ref[...],
            dimension_numbers=(((1,), (1,)), ((), ())),
            preferred_element_type=jnp.float32,
        )

        @pl.when(w == nw - 1)
        def _finish():
            out_ref[...] = acc_ref[...] + bd_ref[...]


def kernel(x, Ae, be, Ad, bd):
    b, dimin = x.shape
    width = Ae.shape[0]
    rb = 256
    wb = 512
    nr = b // rb
    nw = width // wb

    grid = (nr, 2, nw)
    return pl.pallas_call(
        functools.partial(_body, nw=nw, wb=wb),
        grid=grid,
        in_specs=[
            pl.BlockSpec((rb, dimin), lambda r, p, w: (r, 0)),
            pl.BlockSpec((wb, dimin),
                         lambda r, p, w: (jnp.where(p == 0, w, nw - 1), 0)),
            pl.BlockSpec((dimin, wb),
                         lambda r, p, w: (0, jnp.where(p == 1, w, 0))),
            pl.BlockSpec((1, wb),
                         lambda r, p, w: (0, jnp.where(p == 0, w, nw - 1))),
            pl.BlockSpec((1, dimin), lambda r, p, w: (0, 0)),
        ],
        out_specs=pl.BlockSpec((rb, dimin), lambda r, p, w: (r, 0)),
        out_shape=jax.ShapeDtypeStruct((b, dimin), jnp.float32),
        scratch_shapes=[
            pltpu.VMEM((rb, width), jnp.uint32),
            pltpu.VMEM((rb, 1), jnp.uint32),
            pltpu.VMEM((rb, dimin), jnp.float32),
        ],
        compiler_params=pltpu.CompilerParams(
            dimension_semantics=("arbitrary", "arbitrary", "arbitrary"),
        ),
    )(x, Ae, Ad, be, bd)


# speedup vs baseline: 6.2242x; 1.0064x over previous
import functools

import jax
import jax.numpy as jnp
from jax.experimental import pallas as pl
from jax.experimental.pallas import tpu as pltpu

K = 64


def _to_key(h):
    u = jax.lax.bitcast_convert_type(h, jnp.uint32)
    return u ^ jnp.where(h < 0, jnp.uint32(0xFFFFFFFF), jnp.uint32(0x80000000))


def _body(x_ref, ae_ref, ad_ref, be_ref, bd_ref, out_ref, key_ref, thr_ref,
          acc_ref, *, nw, wb):
    p = pl.program_id(1)
    w = pl.program_id(2)

    @pl.when(p == 0)
    def _encode():
        xc = x_ref[...] - bd_ref[...]
        h = jax.lax.dot_general(
            xc, ae_ref[...],
            dimension_numbers=(((1,), (1,)), ((), ())),
            preferred_element_type=jnp.float32,
        ) + be_ref[...]
        key_ref[:, pl.ds(w * wb, wb)] = _to_key(h)

    @pl.when((p == 1) & (w == 0))
    def _threshold():
        keys = key_ref[...]
        t = jnp.zeros((keys.shape[0], 1), jnp.uint32)
        for b in range(31, -1, -1):
            cand = t | jnp.uint32(1 << b)
            cnt = jnp.sum((keys >= cand).astype(jnp.int32), axis=1,
                          keepdims=True)
            t = jnp.where(cnt >= K, cand, t)
        thr_ref[...] = t

    @pl.when(p == 1)
    def _decode():
        @pl.when(w == 0)
        def _init():
            acc_ref[...] = jnp.zeros_like(acc_ref)

        keys = key_ref[:, pl.ds(w * wb, wb)]
        t = thr_ref[...]
        sel = (keys >= t) & (keys > jnp.uint32(0x80000000))
        vals = jax.lax.bitcast_convert_type(keys ^ jnp.uint32(0x80000000),
                                            jnp.float32)
        codes = jnp.where(sel, vals, 0.0).astype(jnp.bfloat16)
        acc_ref[...] += jax.lax.dot_general(
            codes, ad_ref[...],
            dimension_numbers=(((1,), (1,)), ((), ())),
            preferred_element_type=jnp.float32,
        )

        @pl.when(w == nw - 1)
        def _finish():
            out_ref[...] = acc_ref[...] + bd_ref[...]


def kernel(x, Ae, be, Ad, bd):
    b, dimin = x.shape
    width = Ae.shape[0]
    rb = 256
    wb = 512
    nr = b // rb
    nw = width // wb

    grid = (nr, 2, nw)
    return pl.pallas_call(
        functools.partial(_body, nw=nw, wb=wb),
        grid=grid,
        in_specs=[
            pl.BlockSpec((rb, dimin), lambda r, p, w: (r, 0)),
            pl.BlockSpec((wb, dimin),
                         lambda r, p, w: (jnp.where(p == 0, w, nw - 1), 0)),
            pl.BlockSpec((dimin, wb),
                         lambda r, p, w: (0, jnp.where(p == 1, w, 0))),
            pl.BlockSpec((1, wb),
                         lambda r, p, w: (0, jnp.where(p == 0, w, nw - 1))),
            pl.BlockSpec((1, dimin), lambda r, p, w: (0, 0)),
        ],
        out_specs=pl.BlockSpec((rb, dimin), lambda r, p, w: (r, 0)),
        out_shape=jax.ShapeDtypeStruct((b, dimin), jnp.float32),
        scratch_shapes=[
            pltpu.VMEM((rb, width), jnp.uint32),
            pltpu.VMEM((rb, 1), jnp.uint32),
            pltpu.VMEM((rb, dimin), jnp.float32),
        ],
        compiler_params=pltpu.CompilerParams(
            dimension_semantics=("arbitrary", "arbitrary", "arbitrary"),
        ),
    )(x, Ae, Ad.astype(jnp.bfloat16), be, bd)
